# initial kernel scaffold (unmeasured)
import jax
import jax.numpy as jnp
from jax import lax
from jax.experimental import pallas as pl
from jax.experimental.pallas import tpu as pltpu

N_DEV = 8
E4M3_MAX = 448.0
E4M3_MIN_NORMAL = 2.0 ** -6

_CompilerParams = getattr(pltpu, "CompilerParams", None) or getattr(
    pltpu, "TPUCompilerParams"
)
_DeviceIdType = getattr(pl, "DeviceIdType", None) or getattr(pltpu, "DeviceIdType")
MESH = _DeviceIdType.MESH


def _snap_e4m3(v):
    a = jnp.abs(v)
    u = lax.bitcast_convert_type(v, jnp.uint32)
    rb = (u >> jnp.uint32(20)) & jnp.uint32(1)
    un = (u + jnp.uint32(0x7FFFF) + rb) & jnp.uint32(0xFFF00000)
    norm = lax.bitcast_convert_type(un, jnp.float32)
    sub = jnp.round(v * 512.0) * (1.0 / 512.0)
    out = jnp.where(a >= E4M3_MIN_NORMAL, norm, sub)
    return jnp.clip(out, -E4M3_MAX, E4M3_MAX)


def kernel(x, w_mat):
    m_per, k = x.shape
    _, n = w_mat.shape
    n_per = n // N_DEV
    n_half = n_per // 2
    n_steps = 2 * N_DEV

    xb = x.astype(jnp.bfloat16)
    perm = lax.rem(
        lax.axis_index("i") + jnp.arange(N_DEV, dtype=jnp.int32), N_DEV
    )

    def body(
        perm_ref,
        x_ref,
        w_ref,
        out_ref,
        send_buf,
        recv_buf,
        acc_amax,
        amax_buf,
        send_sems,
        recv_sems,
        amax_send_sems,
        amax_recv_sems,
    ):
        s = pl.program_id(0)
        my = lax.axis_index("i")
        half = lax.rem(s, 2)
        target = perm_ref[s // 2]

        @pl.when(s == 0)
        def _init():
            acc_amax[...] = jnp.zeros_like(acc_amax)
            amax_buf[...] = jnp.zeros_like(amax_buf)
            barrier = pltpu.get_barrier_semaphore()
            for t in range(1, N_DEV):
                pl.semaphore_signal(
                    barrier,
                    inc=1,
                    device_id=(lax.rem(my + t, N_DEV),),
                    device_id_type=MESH,
                )
            pl.semaphore_wait(barrier, N_DEV - 1)

        y = jnp.dot(
            x_ref[...],
            w_ref[...].astype(jnp.bfloat16),
            preferred_element_type=jnp.float32,
        )
        acc_amax[...] = jnp.maximum(acc_amax[...], jnp.max(jnp.abs(y)))
        yb = y.astype(jnp.bfloat16)

        @pl.when(s < 2)
        def _own():
            pl.store(
                recv_buf,
                (pl.ds(my * 2 + half, 1), slice(None), slice(None)),
                yb[None],
            )

        @pl.when(s >= 2)
        def _send():
            pl.store(
                send_buf, (pl.ds(s, 1), slice(None), slice(None)), yb[None]
            )
            pltpu.make_async_remote_copy(
                src_ref=send_buf.at[s],
                dst_ref=recv_buf.at[my * 2 + half],
                send_sem=send_sems.at[s],
                recv_sem=recv_sems.at[my * 2 + half],
                device_id=(target,),
                device_id_type=MESH,
            ).start()

        @pl.when(s == n_steps - 1)
        def _finish():
            for t in range(1, N_DEV):
                pltpu.make_async_remote_copy(
                    src_ref=acc_amax,
                    dst_ref=amax_buf.at[my],
                    send_sem=amax_send_sems.at[t],
                    recv_sem=amax_recv_sems.at[my],
                    device_id=(lax.rem(my + t, N_DEV),),
                    device_id_type=MESH,
                ).start()
            for t in range(2, n_steps):
                pltpu.make_async_remote_copy(
                    src_ref=send_buf.at[t],
                    dst_ref=send_buf.at[t],
                    send_sem=send_sems.at[t],
                    recv_sem=recv_sems.at[0],
                    device_id=(my,),
                    device_id_type=MESH,
                ).wait_send()
            for t in range(1, N_DEV):
                pltpu.make_async_remote_copy(
                    src_ref=acc_amax,
                    dst_ref=acc_amax,
                    send_sem=amax_send_sems.at[t],
                    recv_sem=amax_recv_sems.at[0],
                    device_id=(my,),
                    device_id_type=MESH,
                ).wait_send()
            for t in range(1, N_DEV):
                src = lax.rem(my + t, N_DEV)
                for h in range(2):
                    pltpu.make_async_remote_copy(
                        src_ref=recv_buf.at[src * 2 + h],
                        dst_ref=recv_buf.at[src * 2 + h],
                        send_sem=send_sems.at[0],
                        recv_sem=recv_sems.at[src * 2 + h],
                        device_id=(my,),
                        device_id_type=MESH,
                    ).wait_recv()
                pltpu.make_async_remote_copy(
                    src_ref=amax_buf.at[src],
                    dst_ref=amax_buf.at[src],
                    send_sem=amax_send_sems.at[0],
                    recv_sem=amax_recv_sems.at[src],
                    device_id=(my,),
                    device_id_type=MESH,
                ).wait_recv()
            amax = jnp.maximum(jnp.max(amax_buf[...]), jnp.max(acc_amax[...]))
            scale = amax / E4M3_MAX
            inv = E4M3_MAX / amax
            for p in range(N_DEV):
                for h in range(2):
                    blk = recv_buf[p * 2 + h].astype(jnp.float32)
                    out_ref[
                        p * m_per : (p + 1) * m_per,
                        h * n_half : (h + 1) * n_half,
                    ] = _snap_e4m3(blk * inv) * scale

    grid_spec = pltpu.PrefetchScalarGridSpec(
        num_scalar_prefetch=1,
        grid=(n_steps,),
        in_specs=[
            pl.BlockSpec((m_per, k), lambda s, p: (0, 0)),
            pl.BlockSpec((k, n_half), lambda s, p: (0, p[s // 2] * 2 + s % 2)),
        ],
        out_specs=pl.BlockSpec((N_DEV * m_per, n_per), lambda s, p: (0, 0)),
        scratch_shapes=[
            pltpu.VMEM((n_steps, m_per, n_half), jnp.bfloat16),
            pltpu.VMEM((n_steps, m_per, n_half), jnp.bfloat16),
            pltpu.VMEM((8, 128), jnp.float32),
            pltpu.VMEM((N_DEV, 8, 128), jnp.float32),
            pltpu.SemaphoreType.DMA((n_steps,)),
            pltpu.SemaphoreType.DMA((n_steps,)),
            pltpu.SemaphoreType.DMA((N_DEV,)),
            pltpu.SemaphoreType.DMA((N_DEV,)),
        ],
    )
    return pl.pallas_call(
        body,
        grid_spec=grid_spec,
        out_shape=jax.ShapeDtypeStruct((N_DEV * m_per, n_per), jnp.float32),
        compiler_params=_CompilerParams(
            dimension_semantics=("arbitrary",), collective_id=0
        ),
    )(perm, xb, w_mat)


# baseline (device time: 114496 ns/iter reference)
import jax
import jax.numpy as jnp
from jax import lax
from jax.experimental import pallas as pl
from jax.experimental.pallas import tpu as pltpu

N_DEV = 8
E4M3_MAX = 448.0
E4M3_MIN_NORMAL = 2.0 ** -6

_CompilerParams = getattr(pltpu, "CompilerParams", None) or getattr(
    pltpu, "TPUCompilerParams"
)
_DeviceIdType = getattr(pl, "DeviceIdType", None) or getattr(pltpu, "DeviceIdType")
MESH = _DeviceIdType.MESH


def _snap_e4m3(v):
    a = jnp.abs(v)
    u = lax.bitcast_convert_type(v, jnp.uint32)
    rb = (u >> jnp.uint32(20)) & jnp.uint32(1)
    un = (u + jnp.uint32(0x7FFFF) + rb) & jnp.uint32(0xFFF00000)
    norm = lax.bitcast_convert_type(un, jnp.float32)
    sub = jnp.round(v * 512.0) * (1.0 / 512.0)
    out = jnp.where(a >= E4M3_MIN_NORMAL, norm, sub)
    return jnp.clip(out, -E4M3_MAX, E4M3_MAX)


def kernel(x, w_mat):
    m_per, k = x.shape
    _, n = w_mat.shape
    n_per = n // N_DEV
    n_half = n_per // 2
    n_steps = 2 * N_DEV

    xb = x.astype(jnp.bfloat16)
    perm = lax.rem(
        lax.axis_index("i") + jnp.arange(N_DEV, dtype=jnp.int32), N_DEV
    )

    def body(
        perm_ref,
        x_ref,
        w_ref,
        out_ref,
        send_buf,
        recv_buf,
        acc_amax,
        amax_buf,
        send_sems,
        recv_sems,
        amax_send_sems,
        amax_recv_sems,
    ):
        s = pl.program_id(0)
        my = lax.axis_index("i")
        half = lax.rem(s, 2)
        target = perm_ref[s // 2]

        @pl.when(s == 0)
        def _init():
            acc_amax[...] = jnp.zeros_like(acc_amax)
            amax_buf[...] = jnp.zeros_like(amax_buf)
            barrier = pltpu.get_barrier_semaphore()
            for t in range(1, N_DEV):
                pl.semaphore_signal(
                    barrier,
                    inc=1,
                    device_id=(lax.rem(my + t, N_DEV),),
                    device_id_type=MESH,
                )
            pl.semaphore_wait(barrier, N_DEV - 1)

        y = jnp.dot(
            x_ref[...],
            w_ref[...].astype(jnp.bfloat16),
            preferred_element_type=jnp.float32,
        )
        acc_amax[...] = jnp.maximum(acc_amax[...], jnp.max(jnp.abs(y)))
        yb = y.astype(jnp.bfloat16)

        @pl.when(s < 2)
        def _own():
            recv_buf[pl.ds(my * 2 + half, 1)] = yb[None]

        @pl.when(s >= 2)
        def _send():
            send_buf[pl.ds(s, 1)] = yb[None]
            pltpu.make_async_remote_copy(
                src_ref=send_buf.at[s],
                dst_ref=recv_buf.at[my * 2 + half],
                send_sem=send_sems.at[s],
                recv_sem=recv_sems.at[my * 2 + half],
                device_id=(target,),
                device_id_type=MESH,
            ).start()

        @pl.when(s == n_steps - 1)
        def _finish():
            for t in range(1, N_DEV):
                pltpu.make_async_remote_copy(
                    src_ref=acc_amax,
                    dst_ref=amax_buf.at[my],
                    send_sem=amax_send_sems.at[t],
                    recv_sem=amax_recv_sems.at[my],
                    device_id=(lax.rem(my + t, N_DEV),),
                    device_id_type=MESH,
                ).start()
            for t in range(2, n_steps):
                pltpu.make_async_remote_copy(
                    src_ref=send_buf.at[t],
                    dst_ref=send_buf.at[t],
                    send_sem=send_sems.at[t],
                    recv_sem=recv_sems.at[0],
                    device_id=(my,),
                    device_id_type=MESH,
                ).wait_send()
            for t in range(1, N_DEV):
                pltpu.make_async_remote_copy(
                    src_ref=acc_amax,
                    dst_ref=acc_amax,
                    send_sem=amax_send_sems.at[t],
                    recv_sem=amax_recv_sems.at[0],
                    device_id=(my,),
                    device_id_type=MESH,
                ).wait_send()
            for t in range(1, N_DEV):
                src = lax.rem(my + t, N_DEV)
                for h in range(2):
                    pltpu.make_async_remote_copy(
                        src_ref=recv_buf.at[src * 2 + h],
                        dst_ref=recv_buf.at[src * 2 + h],
                        send_sem=send_sems.at[0],
                        recv_sem=recv_sems.at[src * 2 + h],
                        device_id=(my,),
                        device_id_type=MESH,
                    ).wait_recv()
                pltpu.make_async_remote_copy(
                    src_ref=amax_buf.at[src],
                    dst_ref=amax_buf.at[src],
                    send_sem=amax_send_sems.at[0],
                    recv_sem=amax_recv_sems.at[src],
                    device_id=(my,),
                    device_id_type=MESH,
                ).wait_recv()
            amax = jnp.maximum(jnp.max(amax_buf[...]), jnp.max(acc_amax[...]))
            scale = amax / E4M3_MAX
            inv = E4M3_MAX / amax
            for p in range(N_DEV):
                for h in range(2):
                    blk = recv_buf[p * 2 + h].astype(jnp.float32)
                    out_ref[
                        p * m_per : (p + 1) * m_per,
                        h * n_half : (h + 1) * n_half,
                    ] = (_snap_e4m3(blk * inv) * scale).astype(jnp.bfloat16)

    grid_spec = pltpu.PrefetchScalarGridSpec(
        num_scalar_prefetch=1,
        grid=(n_steps,),
        in_specs=[
            pl.BlockSpec((m_per, k), lambda s, p: (0, 0)),
            pl.BlockSpec((k, n_half), lambda s, p: (0, p[s // 2] * 2 + s % 2)),
        ],
        out_specs=pl.BlockSpec((N_DEV * m_per, n_per), lambda s, p: (0, 0)),
        scratch_shapes=[
            pltpu.VMEM((n_steps, m_per, n_half), jnp.bfloat16),
            pltpu.VMEM((n_steps, m_per, n_half), jnp.bfloat16),
            pltpu.VMEM((8, 128), jnp.float32),
            pltpu.VMEM((N_DEV, 8, 128), jnp.float32),
            pltpu.SemaphoreType.DMA((n_steps,)),
            pltpu.SemaphoreType.DMA((n_steps,)),
            pltpu.SemaphoreType.DMA((N_DEV,)),
            pltpu.SemaphoreType.DMA((N_DEV,)),
        ],
    )
    return pl.pallas_call(
        body,
        grid_spec=grid_spec,
        out_shape=jax.ShapeDtypeStruct((N_DEV * m_per, n_per), jnp.bfloat16),
        compiler_params=_CompilerParams(
            dimension_semantics=("arbitrary",),
            collective_id=0,
            vmem_limit_bytes=100 * 1024 * 1024,
        ),
    )(perm, xb, w_mat)


# device time: 101031 ns/iter; 1.1333x vs baseline; 1.1333x over previous
import jax
import jax.numpy as jnp
from jax import lax
from jax.experimental import pallas as pl
from jax.experimental.pallas import tpu as pltpu

N_DEV = 8
E4M3_MAX = 448.0

_CompilerParams = getattr(pltpu, "CompilerParams", None) or getattr(
    pltpu, "TPUCompilerParams"
)
_DeviceIdType = getattr(pl, "DeviceIdType", None) or getattr(pltpu, "DeviceIdType")
MESH = _DeviceIdType.MESH


def _snap_e4m3(v):
    u = lax.bitcast_convert_type(v, jnp.uint32)
    rb = (u >> jnp.uint32(20)) & jnp.uint32(1)
    un = (u + jnp.uint32(0x7FFFF) + rb) & jnp.uint32(0xFFF00000)
    norm = lax.bitcast_convert_type(un, jnp.float32)
    return jnp.clip(norm, -E4M3_MAX, E4M3_MAX)


def kernel(x, w_mat):
    m_per, k = x.shape
    _, n = w_mat.shape
    n_per = n // N_DEV
    n_half = n_per // 2
    n_steps = 2 * N_DEV

    xb = x.astype(jnp.bfloat16)
    offsets = jnp.array([1, 2, 3, 4, 5, 6, 7, 0], dtype=jnp.int32)
    perm = lax.rem(lax.axis_index("i") + offsets, N_DEV)

    def body(
        perm_ref,
        x_ref,
        w_ref,
        out_ref,
        wbuf0,
        wbuf1,
        send_buf,
        recv_buf,
        acc_amax,
        amax_buf,
        send_sems,
        recv_sems,
        amax_send_sems,
        amax_recv_sems,
    ):
        s = pl.program_id(0)
        my = lax.axis_index("i")
        even = lax.rem(s, 2) == 0

        @pl.when(s == 0)
        def _init():
            acc_amax[...] = jnp.zeros_like(acc_amax)
            amax_buf[...] = jnp.zeros_like(amax_buf)
            barrier = pltpu.get_barrier_semaphore()
            for t in range(1, N_DEV):
                pl.semaphore_signal(
                    barrier,
                    inc=1,
                    device_id=(lax.rem(my + t, N_DEV),),
                    device_id_type=MESH,
                )
            pl.semaphore_wait(barrier, N_DEV - 1)

        def step_work(wb_ref):
            pi = (s - 1) // 2
            h = lax.rem(s - 1, 2)
            target = perm_ref[pi]
            y = jnp.dot(
                x_ref[...], wb_ref[...], preferred_element_type=jnp.float32
            )
            acc_amax[...] = jnp.maximum(acc_amax[...], jnp.max(jnp.abs(y)))
            yb = y.astype(jnp.bfloat16)

            @pl.when(pi < N_DEV - 1)
            def _send():
                send_buf[pl.ds(s - 1, 1)] = yb[None]
                pltpu.make_async_remote_copy(
                    src_ref=send_buf.at[s - 1],
                    dst_ref=recv_buf.at[my * 2 + h],
                    send_sem=send_sems.at[s - 1],
                    recv_sem=recv_sems.at[my * 2 + h],
                    device_id=(target,),
                    device_id_type=MESH,
                ).start()

            @pl.when(pi == N_DEV - 1)
            def _own():
                recv_buf[pl.ds(my * 2 + h, 1)] = yb[None]

        @pl.when(even)
        def _even():
            @pl.when(s >= 2)
            def _():
                step_work(wbuf1)

            @pl.when(s <= n_steps - 1)
            def _():
                wbuf0[...] = w_ref[...].astype(jnp.bfloat16)

        @pl.when(jnp.logical_not(even))
        def _odd():
            step_work(wbuf0)
            wbuf1[...] = w_ref[...].astype(jnp.bfloat16)

        @pl.when(s == n_steps)
        def _finish():
            for t in range(1, N_DEV):
                pltpu.make_async_remote_copy(
                    src_ref=acc_amax,
                    dst_ref=amax_buf.at[my],
                    send_sem=amax_send_sems.at[t],
                    recv_sem=amax_recv_sems.at[my],
                    device_id=(lax.rem(my + t, N_DEV),),
                    device_id_type=MESH,
                ).start()
            for t in range(1, N_DEV):
                src = lax.rem(my + t, N_DEV)
                pltpu.make_async_remote_copy(
                    src_ref=amax_buf.at[src],
                    dst_ref=amax_buf.at[src],
                    send_sem=amax_send_sems.at[0],
                    recv_sem=amax_recv_sems.at[src],
                    device_id=(my,),
                    device_id_type=MESH,
                ).wait_recv()
            amax = jnp.maximum(jnp.max(amax_buf[...]), jnp.max(acc_amax[...]))
            scale = amax / E4M3_MAX
            inv = E4M3_MAX / amax

            def dequant_store(src):
                for h2 in range(2):
                    blk = recv_buf[pl.ds(src * 2 + h2, 1)][0].astype(
                        jnp.float32
                    )
                    q = _snap_e4m3(blk * inv)
                    out_ref[
                        pl.ds(src * m_per, m_per),
                        h2 * n_half : (h2 + 1) * n_half,
                    ] = (q * scale).astype(jnp.bfloat16)

            dequant_store(my)
            for t in range(1, N_DEV):
                src = lax.rem(my - t + N_DEV, N_DEV)
                for h2 in range(2):
                    pltpu.make_async_remote_copy(
                        src_ref=recv_buf.at[src * 2 + h2],
                        dst_ref=recv_buf.at[src * 2 + h2],
                        send_sem=send_sems.at[0],
                        recv_sem=recv_sems.at[src * 2 + h2],
                        device_id=(my,),
                        device_id_type=MESH,
                    ).wait_recv()
                dequant_store(src)
            for t in range(2 * (N_DEV - 1)):
                pltpu.make_async_remote_copy(
                    src_ref=send_buf.at[t],
                    dst_ref=send_buf.at[t],
                    send_sem=send_sems.at[t],
                    recv_sem=recv_sems.at[0],
                    device_id=(my,),
                    device_id_type=MESH,
                ).wait_send()
            for t in range(1, N_DEV):
                pltpu.make_async_remote_copy(
                    src_ref=acc_amax,
                    dst_ref=acc_amax,
                    send_sem=amax_send_sems.at[t],
                    recv_sem=amax_recv_sems.at[0],
                    device_id=(my,),
                    device_id_type=MESH,
                ).wait_send()

    grid_spec = pltpu.PrefetchScalarGridSpec(
        num_scalar_prefetch=1,
        grid=(n_steps + 1,),
        in_specs=[
            pl.BlockSpec((m_per, k), lambda s, p: (0, 0)),
            pl.BlockSpec(
                (k, n_half),
                lambda s, p: (
                    0,
                    p[jnp.minimum(s, 15) // 2] * 2 + jnp.minimum(s, 15) % 2,
                ),
            ),
        ],
        out_specs=pl.BlockSpec((N_DEV * m_per, n_per), lambda s, p: (0, 0)),
        scratch_shapes=[
            pltpu.VMEM((k, n_half), jnp.bfloat16),
            pltpu.VMEM((k, n_half), jnp.bfloat16),
            pltpu.VMEM((2 * (N_DEV - 1), m_per, n_half), jnp.bfloat16),
            pltpu.VMEM((2 * N_DEV, m_per, n_half), jnp.bfloat16),
            pltpu.VMEM((8, 128), jnp.float32),
            pltpu.VMEM((N_DEV, 8, 128), jnp.float32),
            pltpu.SemaphoreType.DMA((2 * (N_DEV - 1),)),
            pltpu.SemaphoreType.DMA((2 * N_DEV,)),
            pltpu.SemaphoreType.DMA((N_DEV,)),
            pltpu.SemaphoreType.DMA((N_DEV,)),
        ],
    )
    return pl.pallas_call(
        body,
        grid_spec=grid_spec,
        out_shape=jax.ShapeDtypeStruct((N_DEV * m_per, n_per), jnp.bfloat16),
        compiler_params=_CompilerParams(
            dimension_semantics=("arbitrary",),
            collective_id=0,
            vmem_limit_bytes=100 * 1024 * 1024,
        ),
    )(perm, xb, w_mat)
